# baseline (speedup 1.0000x reference)
import functools

import jax
import jax.numpy as jnp
from jax import lax
from jax.experimental import pallas as pl
from jax.experimental.pallas import tpu as pltpu
from jax.experimental.pallas import tpu_sc as plsc

LANES = 128
CH = 512
KR = CH // LANES
NCLS = 28


def _pad_up(v, m):
    return (v + m - 1) // m * m


def _make_deg_kernel(Ep, Np):
    rows_total = Ep // LANES
    rows_per_tile = rows_total // 32
    n_chunks = rows_per_tile // KR
    seg = Np // 16

    mesh = plsc.VectorSubcoreMesh(core_axis_name="c", subcore_axis_name="s")

    @functools.partial(
        pl.kernel,
        out_type=jax.ShapeDtypeStruct((2, Np), jnp.float32),
        mesh=mesh,
        scratch_types=[
            pltpu.VMEM_SHARED((Np,), jnp.float32),
            pltpu.VMEM((2, KR, LANES), jnp.int32),
            pltpu.VMEM((KR, LANES), jnp.float32),
            pltpu.SemaphoreType.DMA,
            pltpu.SemaphoreType.DMA,
        ],
    )
    def deg_kernel(dst_hbm, zeros_hbm, ones_hbm, out_hbm, deg_sh, dst_v, ones_v,
                   sem_l, sem_sc):
        c = lax.axis_index("c")
        s = lax.axis_index("s")
        wid = c * 16 + s
        pltpu.sync_copy(zeros_hbm.at[pl.ds(0, seg)], deg_sh.at[pl.ds(s * seg, seg)])
        pltpu.sync_copy(ones_hbm, ones_v)
        plsc.subcore_barrier()

        base_row = wid * rows_per_tile

        def fire_lin(i, b):
            row0 = base_row + i * KR
            pltpu.async_copy(dst_hbm.at[pl.ds(row0, KR), :], dst_v.at[b], sem_l)

        def wait_lin(b):
            pltpu.make_async_copy(dst_hbm.at[pl.ds(0, KR), :], dst_v.at[b],
                                  sem_l).wait()

        fire_lin(0, 0)

        def dstep(i, b):
            wait_lin(b)

            @pl.when(i + 1 < n_chunks)
            def _pref():
                fire_lin(i + 1, 1 - b)

            scs = [pltpu.async_copy(ones_v.at[j], deg_sh.at[dst_v.at[b, j]],
                                    sem_sc, add=True) for j in range(KR)]
            for d in scs:
                d.wait()

        def body(it2, carry):
            dstep(2 * it2, 0)
            dstep(2 * it2 + 1, 1)
            return carry

        lax.fori_loop(0, n_chunks // 2, body, 0)
        plsc.subcore_barrier()
        pltpu.sync_copy(deg_sh.at[pl.ds(s * seg, seg)],
                        out_hbm.at[c, pl.ds(s * seg, seg)])

    return deg_kernel


def _make_main_kernel(Ep, Np):
    H = Np // 2
    A_SIZE = H * NCLS
    rows_total = Ep // LANES
    rows_per_tile = rows_total // 16
    n_chunks = rows_per_tile // KR
    a_seg = (A_SIZE + 2048) // 16
    a_out_seg = A_SIZE // 16
    r_seg = (Np + 2048) // 16
    r_out_seg = Np // 16

    mesh = plsc.VectorSubcoreMesh(core_axis_name="c", subcore_axis_name="s")

    @functools.partial(
        pl.kernel,
        out_type=(
            jax.ShapeDtypeStruct((2, A_SIZE), jnp.float32),
            jax.ShapeDtypeStruct((2, Np), jnp.float32),
        ),
        mesh=mesh,
        scratch_types=[
            pltpu.VMEM_SHARED((A_SIZE + 2048,), jnp.float32),
            pltpu.VMEM_SHARED((Np + 2048,), jnp.float32),
            pltpu.VMEM((3, KR, LANES), jnp.int32),
            pltpu.VMEM((3, KR, LANES), jnp.int32),
            pltpu.VMEM((3, KR, LANES), jnp.float32),
            pltpu.VMEM((3, KR, LANES), jnp.float32),
            pltpu.VMEM((3, KR, LANES), jnp.int32),
            pltpu.VMEM((3, KR, LANES), jnp.int32),
            pltpu.SemaphoreType.DMA,
            pltpu.SemaphoreType.DMA,
            pltpu.SemaphoreType.DMA,
            pltpu.SemaphoreType.DMA,
            pltpu.SemaphoreType.DMA,
            pltpu.SemaphoreType.DMA,
            pltpu.SemaphoreType.DMA,
        ],
    )
    def main_kernel(src_hbm, dst_hbm, q_hbm, zeros_hbm,
                    a_out, r_out,
                    a_sh, r_sh, src_v, dst_v, dvs_v, dvd_v,
                    binA_v, binR_v, sem_l,
                    sem_g0, sem_g1, sem_g2, sem_s0, sem_s1, sem_s2):
        c = lax.axis_index("c")
        s = lax.axis_index("s")
        pltpu.sync_copy(zeros_hbm.at[pl.ds(0, a_seg)], a_sh.at[pl.ds(s * a_seg, a_seg)])
        pltpu.sync_copy(zeros_hbm.at[pl.ds(0, r_seg)], r_sh.at[pl.ds(s * r_seg, r_seg)])
        plsc.subcore_barrier()

        base_row = s * rows_per_tile
        half_base = c * H
        sem_g = [sem_g0, sem_g1, sem_g2]
        sem_s = [sem_s0, sem_s1, sem_s2]

        def fire_linear(i, g):
            row0 = base_row + i * KR
            pltpu.async_copy(src_hbm.at[pl.ds(row0, KR), :], src_v.at[g], sem_l)
            pltpu.async_copy(dst_hbm.at[pl.ds(row0, KR), :], dst_v.at[g], sem_l)

        def wait_linear(g):
            pltpu.make_async_copy(src_hbm.at[pl.ds(0, KR), :], src_v.at[g], sem_l).wait()
            pltpu.make_async_copy(dst_hbm.at[pl.ds(0, KR), :], dst_v.at[g], sem_l).wait()

        def fire_gathers(g, p2):
            for j in range(KR):
                pltpu.async_copy(q_hbm.at[src_v.at[g, j]], dvs_v.at[g, j], sem_g[g])

            @pl.when(c == p2)
            def _rpart():
                for j in range(KR):
                    pltpu.async_copy(q_hbm.at[dst_v.at[g, j]], dvd_v.at[g, j],
                                     sem_g[g])

        def wait_gathers(g, p2):
            for j in range(KR):
                pltpu.make_async_copy(q_hbm.at[src_v.at[g, j]], dvs_v.at[g, j],
                                      sem_g[g]).wait()

            @pl.when(c == p2)
            def _rpart():
                for j in range(KR):
                    pltpu.make_async_copy(q_hbm.at[dst_v.at[g, j]], dvd_v.at[g, j],
                                          sem_g[g]).wait()

        def fire_scatters(g, p2):
            for j in range(KR):
                pltpu.async_copy(dvs_v.at[g, j], a_sh.at[binA_v.at[g, j]],
                                 sem_s[g], add=True)

            @pl.when(c == p2)
            def _rpart():
                for j in range(KR):
                    pltpu.async_copy(dvd_v.at[g, j], r_sh.at[binR_v.at[g, j]],
                                     sem_s[g], add=True)

        def wait_scatters(g, p2):
            for j in range(KR):
                pltpu.make_async_copy(dvs_v.at[g, j], a_sh.at[binA_v.at[g, j]],
                                      sem_s[g]).wait()

            @pl.when(c == p2)
            def _rpart():
                for j in range(KR):
                    pltpu.make_async_copy(dvd_v.at[g, j], r_sh.at[binR_v.at[g, j]],
                                          sem_s[g]).wait()

        fire_linear(0, 0)
        wait_linear(0)
        fire_gathers(0, 0)
        fire_linear(1, 1)

        def step(i, u):
            p2, p3 = u % 2, u % 3
            p2n, p3n = (u + 1) % 2, (u + 1) % 3
            p2p, p3p = (u + 5) % 2, (u + 5) % 3

            @pl.when(i + 1 < n_chunks)
            def _next():
                wait_linear(p3n)
                fire_gathers(p3n, p2n)

            wait_gathers(p3, p2)

            @pl.when(i >= 1)
            def _drain():
                wait_scatters(p3p, p2p)

            for j in range(KR):
                for t in range(LANES // 16):
                    sl = pl.ds(t * 16, 16)
                    d16 = dst_v[p3, j, sl]
                    qs = dvs_v[p3, j, sl]
                    c16 = (qs * 0.5).astype(jnp.int32)
                    dvs_v[p3, j, sl] = qs - 2.0 * c16.astype(jnp.float32)
                    rel = d16 - half_base
                    ok = (rel >= 0) & (rel < H)
                    binA_v[p3, j, sl] = jnp.where(ok, c16 * H + rel, A_SIZE)

            @pl.when(c == p2)
            def _rdecode():
                for j in range(KR):
                    for t in range(LANES // 16):
                        sl = pl.ds(t * 16, 16)
                        qd = dvd_v[p3, j, sl]
                        dvd_v[p3, j, sl] = (
                            qd - 2.0 * (qd * 0.5).astype(jnp.int32).astype(jnp.float32))
                        binR_v[p3, j, sl] = src_v[p3, j, sl]

            fire_scatters(p3, p2)

            @pl.when(i + 2 < n_chunks)
            def _pref():
                fire_linear(i + 2, (u + 2) % 3)

        def body(it6, carry):
            i0 = 6 * it6
            for u in range(6):
                step(i0 + u, u)
            return carry

        lax.fori_loop(0, n_chunks // 6, body, 0)
        wait_scatters((n_chunks - 1) % 3, (n_chunks - 1) % 2)
        plsc.subcore_barrier()
        pltpu.sync_copy(a_sh.at[pl.ds(s * a_out_seg, a_out_seg)],
                        a_out.at[c, pl.ds(s * a_out_seg, a_out_seg)])
        pltpu.sync_copy(r_sh.at[pl.ds(s * r_out_seg, r_out_seg)],
                        r_out.at[c, pl.ds(s * r_out_seg, r_out_seg)])

    return main_kernel


def _rsqrt_body(deg_ref, cls_ref, dinv_ref, q_ref):
    d = deg_ref[0] + deg_ref[1] + 1.0
    dinv = lax.rsqrt(d)
    dinv_ref[...] = dinv
    q_ref[...] = 2.0 * cls_ref[...].astype(jnp.float32) + dinv


def _dense_body(n_nodes, n_steps, RL, H,
                a0_ref, a1_ref, dinv0_ref, dinv1_ref, x0_ref, x1_ref,
                r_0ref, r_1ref,
                emb_ref, w1_ref, b1_ref, w2_ref, b2_ref, wl_ref, bl_ref,
                out_ref, acc_ref):
    i = pl.program_id(0)

    @pl.when(i == 0)
    def _init():
        acc_ref[...] = jnp.zeros_like(acc_ref)

    t1t = lax.dot_general(w1_ref[...], emb_ref[...],
                          (((0,), (1,)), ((), ())),
                          preferred_element_type=jnp.float32)
    b1c = b1_ref[...]

    def half(a_ref, dv_ref, x_ref, r_ref, masked):
        dv = dv_ref[...]
        xb = x_ref[...]
        at = a_ref[...]
        oh = (xb == lax.broadcasted_iota(jnp.int32, (NCLS, RL), 0)
              ).astype(jnp.float32)
        bt = at + dv * oh
        a1t = dv * jnp.dot(t1t, bt, preferred_element_type=jnp.float32) + b1c
        h1t = jnp.maximum(a1t, 0.0)
        w = dv * (r_ref[0:1, :] + r_ref[1:2, :]) + dv * dv
        if masked:
            glob = lax.broadcasted_iota(jnp.int32, (1, RL), 1) + (H + i * RL)
            w = jnp.where(glob < n_nodes, w, 0.0)
        return jnp.sum(h1t * w, axis=1, keepdims=True)

    acc_ref[...] += (half(a0_ref, dinv0_ref, x0_ref, r_0ref, False)
                     + half(a1_ref, dinv1_ref, x1_ref, r_1ref, True))

    @pl.when(i == n_steps - 1)
    def _head():
        nf = jnp.float32(n_nodes)
        s_t = acc_ref[...]
        e_t = lax.dot_general(w2_ref[...], s_t, (((0,), (0,)), ((), ())),
                              preferred_element_type=jnp.float32) + nf * b2_ref[...]
        out_ref[...] = lax.dot_general(wl_ref[...], e_t, (((0,), (0,)), ((), ())),
                                       preferred_element_type=jnp.float32
                                       ) + nf * bl_ref[...]


def kernel(x, edge_index, edge_attr, emb, W1, b1, W2, b2, Wl, bl):
    n = x.shape[0]
    e = edge_index.shape[1]
    Np = _pad_up(n, 2048)
    Ep = _pad_up(e, 192 * CH)
    H = Np // 2
    A_SIZE = H * NCLS

    src = edge_index[0].astype(jnp.int32)
    dst = edge_index[1].astype(jnp.int32)
    pad_idx = jnp.full((Ep - e,), Np - 1, jnp.int32)
    src_p = jnp.concatenate([src, pad_idx]).reshape(Ep // LANES, LANES)
    dst_p = jnp.concatenate([dst, pad_idx]).reshape(Ep // LANES, LANES)
    xcls = jnp.concatenate(
        [x[:, 0].astype(jnp.int32), jnp.zeros((Np - n,), jnp.int32)])

    zeros_big = jnp.zeros(((A_SIZE + 2048) // 16,), jnp.float32)
    ones_chunk = jnp.ones((KR, LANES), jnp.float32)

    deg2 = _make_deg_kernel(Ep, Np)(dst_p, zeros_big, ones_chunk)

    dinv_2d, q_2d = pl.pallas_call(
        _rsqrt_body,
        out_shape=(jax.ShapeDtypeStruct((Np // 128, 128), jnp.float32),
                   jax.ShapeDtypeStruct((Np // 128, 128), jnp.float32)),
    )(deg2.reshape(2, Np // 128, 128), xcls.reshape(Np // 128, 128))
    dinv = dinv_2d.reshape(Np)
    q_tab = q_2d.reshape(Np)

    a_halves, r2 = _make_main_kernel(Ep, Np)(src_p, dst_p, q_tab, zeros_big)
    a0 = a_halves[0].reshape(NCLS, H)
    a1 = a_halves[1].reshape(NCLS, H)

    n_steps = 8
    RL = H // n_steps
    dinv_row = dinv.reshape(1, Np)
    x_row = xcls.reshape(1, Np)
    lo = lambda i: (0, i)
    hi = lambda i: (0, n_steps + i)
    head = pl.pallas_call(
        functools.partial(_dense_body, n, n_steps, RL, H),
        grid=(n_steps,),
        in_specs=[
            pl.BlockSpec((NCLS, RL), lo),
            pl.BlockSpec((NCLS, RL), lambda i: (0, i)),
            pl.BlockSpec((1, RL), lo),
            pl.BlockSpec((1, RL), hi),
            pl.BlockSpec((1, RL), lo),
            pl.BlockSpec((1, RL), hi),
            pl.BlockSpec((2, RL), lo),
            pl.BlockSpec((2, RL), hi),
            pl.BlockSpec((NCLS, 16), lambda i: (0, 0)),
            pl.BlockSpec((16, 16), lambda i: (0, 0)),
            pl.BlockSpec((16, 1), lambda i: (0, 0)),
            pl.BlockSpec((16, 16), lambda i: (0, 0)),
            pl.BlockSpec((16, 1), lambda i: (0, 0)),
            pl.BlockSpec((16, 1), lambda i: (0, 0)),
            pl.BlockSpec((1, 1), lambda i: (0, 0)),
        ],
        out_specs=pl.BlockSpec((1, 1), lambda i: (0, 0)),
        out_shape=jax.ShapeDtypeStruct((1, 1), jnp.float32),
        scratch_shapes=[pltpu.VMEM((16, 1), jnp.float32)],
    )(a0, a1, dinv_row, dinv_row, x_row, x_row, r2, r2,
      emb, W1, b1.reshape(16, 1), W2, b2.reshape(16, 1), Wl,
      bl.reshape(1, 1))

    return head.reshape(1)


# speedup vs baseline: 1.0850x; 1.0850x over previous
import functools

import jax
import jax.numpy as jnp
from jax import lax
from jax.experimental import pallas as pl
from jax.experimental.pallas import tpu as pltpu
from jax.experimental.pallas import tpu_sc as plsc

LANES = 128
CH = 512
KR = CH // LANES
NCLS = 28


def _pad_up(v, m):
    return (v + m - 1) // m * m


def _make_deg_kernel(Ep, Np):
    rows_total = Ep // LANES
    rows_per_tile = rows_total // 32
    n_chunks = rows_per_tile // KR
    seg = Np // 16

    mesh = plsc.VectorSubcoreMesh(core_axis_name="c", subcore_axis_name="s")

    @functools.partial(
        pl.kernel,
        out_type=jax.ShapeDtypeStruct((2, Np), jnp.float32),
        mesh=mesh,
        scratch_types=[
            pltpu.VMEM_SHARED((Np,), jnp.float32),
            pltpu.VMEM((2, KR, LANES), jnp.int32),
            pltpu.VMEM((KR, LANES), jnp.float32),
            pltpu.SemaphoreType.DMA,
            pltpu.SemaphoreType.DMA,
        ],
    )
    def deg_kernel(dst_hbm, zeros_hbm, ones_hbm, out_hbm, deg_sh, dst_v, ones_v,
                   sem_l, sem_sc):
        c = lax.axis_index("c")
        s = lax.axis_index("s")
        wid = c * 16 + s
        pltpu.sync_copy(zeros_hbm.at[pl.ds(0, seg)], deg_sh.at[pl.ds(s * seg, seg)])
        pltpu.sync_copy(ones_hbm, ones_v)
        plsc.subcore_barrier()

        base_row = wid * rows_per_tile

        def fire_lin(i, b):
            row0 = base_row + i * KR
            pltpu.async_copy(dst_hbm.at[pl.ds(row0, KR), :], dst_v.at[b], sem_l)

        def wait_lin(b):
            pltpu.make_async_copy(dst_hbm.at[pl.ds(0, KR), :], dst_v.at[b],
                                  sem_l).wait()

        fire_lin(0, 0)

        def dstep(i, b):
            wait_lin(b)

            @pl.when(i + 1 < n_chunks)
            def _pref():
                fire_lin(i + 1, 1 - b)

            scs = [pltpu.async_copy(ones_v.at[j], deg_sh.at[dst_v.at[b, j]],
                                    sem_sc, add=True) for j in range(KR)]
            for d in scs:
                d.wait()

        def body(it2, carry):
            dstep(2 * it2, 0)
            dstep(2 * it2 + 1, 1)
            return carry

        lax.fori_loop(0, n_chunks // 2, body, 0)
        plsc.subcore_barrier()
        pltpu.sync_copy(deg_sh.at[pl.ds(s * seg, seg)],
                        out_hbm.at[c, pl.ds(s * seg, seg)])

    return deg_kernel


def _make_main_kernel(Ep, Np):
    H = Np // 2
    A_SIZE = H * NCLS
    rows_total = Ep // LANES
    rows_per_tile = rows_total // 16
    n_chunks = rows_per_tile // KR
    a_seg = (A_SIZE + 2048) // 16
    a_out_seg = A_SIZE // 16
    r_seg = (Np + 2048) // 16
    r_out_seg = Np // 16

    mesh = plsc.VectorSubcoreMesh(core_axis_name="c", subcore_axis_name="s")

    @functools.partial(
        pl.kernel,
        out_type=(
            jax.ShapeDtypeStruct((2, A_SIZE), jnp.float32),
            jax.ShapeDtypeStruct((2, Np), jnp.float32),
        ),
        mesh=mesh,
        scratch_types=[
            pltpu.VMEM_SHARED((A_SIZE + 2048,), jnp.float32),
            pltpu.VMEM_SHARED((Np + 2048,), jnp.float32),
            pltpu.VMEM((2, KR, LANES), jnp.int32),
            pltpu.VMEM((2, KR, LANES), jnp.int32),
            pltpu.VMEM((2, KR, LANES), jnp.float32),
            pltpu.VMEM((2, KR, LANES), jnp.float32),
            pltpu.VMEM((KR, LANES), jnp.int32),
            pltpu.SemaphoreType.DMA,
            pltpu.SemaphoreType.DMA,
            pltpu.SemaphoreType.DMA,
            pltpu.SemaphoreType.DMA,
        ],
    )
    def main_kernel(src_hbm, dst_hbm, q_hbm, zeros_hbm,
                    a_out, r_out,
                    a_sh, r_sh, src_v, dst_v, dvs_v, dvd_v,
                    binA_v, sem_l, sem_g0, sem_g1, sem_sc):
        c = lax.axis_index("c")
        s = lax.axis_index("s")
        pltpu.sync_copy(zeros_hbm.at[pl.ds(0, a_seg)], a_sh.at[pl.ds(s * a_seg, a_seg)])
        pltpu.sync_copy(zeros_hbm.at[pl.ds(0, r_seg)], r_sh.at[pl.ds(s * r_seg, r_seg)])
        plsc.subcore_barrier()

        base_row = s * rows_per_tile
        half_base = c * H
        sem_g = [sem_g0, sem_g1]

        def fire_linear(i, b):
            row0 = base_row + i * KR
            pltpu.async_copy(src_hbm.at[pl.ds(row0, KR), :], src_v.at[b], sem_l)
            pltpu.async_copy(dst_hbm.at[pl.ds(row0, KR), :], dst_v.at[b], sem_l)

        def wait_linear(b):
            pltpu.make_async_copy(src_hbm.at[pl.ds(0, KR), :], src_v.at[b], sem_l).wait()
            pltpu.make_async_copy(dst_hbm.at[pl.ds(0, KR), :], dst_v.at[b], sem_l).wait()

        def fire_gathers(b):
            for j in range(KR):
                pltpu.async_copy(q_hbm.at[src_v.at[b, j]], dvs_v.at[b, j], sem_g[b])

            @pl.when(c == b)
            def _rpart():
                for j in range(KR):
                    pltpu.async_copy(q_hbm.at[dst_v.at[b, j]], dvd_v.at[b, j],
                                     sem_g[b])

        def wait_gathers(b):
            for j in range(KR):
                pltpu.make_async_copy(q_hbm.at[src_v.at[b, j]], dvs_v.at[b, j],
                                      sem_g[b]).wait()

            @pl.when(c == b)
            def _rpart():
                for j in range(KR):
                    pltpu.make_async_copy(q_hbm.at[dst_v.at[b, j]], dvd_v.at[b, j],
                                          sem_g[b]).wait()

        fire_linear(0, 0)
        wait_linear(0)
        fire_gathers(0)
        fire_linear(1, 1)

        def step(i, b):
            @pl.when(i + 1 < n_chunks)
            def _next():
                wait_linear(1 - b)
                fire_gathers(1 - b)
            wait_gathers(b)
            for j in range(KR):
                for t in range(LANES // 16):
                    sl = pl.ds(t * 16, 16)
                    d16 = dst_v[b, j, sl]
                    qs = dvs_v[b, j, sl]
                    c16 = (qs * 0.5).astype(jnp.int32)
                    dvs_v[b, j, sl] = qs - 2.0 * c16.astype(jnp.float32)
                    rel = d16 - half_base
                    ok = rel.astype(jnp.uint32) < jnp.uint32(H)
                    binA_v[j, sl] = jnp.where(ok, c16 * H + rel, A_SIZE)

            @pl.when(c == b)
            def _rdecode():
                for j in range(KR):
                    for t in range(LANES // 16):
                        sl = pl.ds(t * 16, 16)
                        qd = dvd_v[b, j, sl]
                        dvd_v[b, j, sl] = (
                            qd - 2.0 * (qd * 0.5).astype(jnp.int32).astype(jnp.float32))

            @pl.when(i + 2 < n_chunks)
            def _pref():
                fire_linear(i + 2, b)
            scs = []
            for j in range(KR):
                scs.append(pltpu.async_copy(dvs_v.at[b, j], a_sh.at[binA_v.at[j]],
                                            sem_sc, add=True))
            for sc_d in scs:
                sc_d.wait()

            @pl.when(c == b)
            def _rscatter():
                rs = []
                for j in range(KR):
                    rs.append(pltpu.async_copy(dvd_v.at[b, j], r_sh.at[src_v.at[b, j]],
                                               sem_sc, add=True))
                for rd in rs:
                    rd.wait()

        def body(it2, carry):
            step(2 * it2, 0)
            step(2 * it2 + 1, 1)
            return carry

        lax.fori_loop(0, n_chunks // 2, body, 0)
        plsc.subcore_barrier()
        pltpu.sync_copy(a_sh.at[pl.ds(s * a_out_seg, a_out_seg)],
                        a_out.at[c, pl.ds(s * a_out_seg, a_out_seg)])
        pltpu.sync_copy(r_sh.at[pl.ds(s * r_out_seg, r_out_seg)],
                        r_out.at[c, pl.ds(s * r_out_seg, r_out_seg)])

    return main_kernel


def _rsqrt_body(deg_ref, cls_ref, dinv_ref, q_ref):
    d = deg_ref[0] + deg_ref[1] + 1.0
    dinv = lax.rsqrt(d)
    dinv_ref[...] = dinv
    q_ref[...] = 2.0 * cls_ref[...].astype(jnp.float32) + dinv


def _dense_body(n_nodes, n_steps, RL, H,
                a0_ref, a1_ref, dinv0_ref, dinv1_ref, x0_ref, x1_ref,
                r_0ref, r_1ref,
                emb_ref, w1_ref, b1_ref, w2_ref, b2_ref, wl_ref, bl_ref,
                out_ref, acc_ref):
    i = pl.program_id(0)

    @pl.when(i == 0)
    def _init():
        acc_ref[...] = jnp.zeros_like(acc_ref)

    t1t = lax.dot_general(w1_ref[...], emb_ref[...],
                          (((0,), (1,)), ((), ())),
                          preferred_element_type=jnp.float32)
    b1c = b1_ref[...]

    def half(a_ref, dv_ref, x_ref, r_ref, masked):
        dv = dv_ref[...]
        xb = x_ref[...]
        at = a_ref[...]
        oh = (xb == lax.broadcasted_iota(jnp.int32, (NCLS, RL), 0)
              ).astype(jnp.float32)
        bt = at + dv * oh
        a1t = dv * jnp.dot(t1t, bt, preferred_element_type=jnp.float32) + b1c
        h1t = jnp.maximum(a1t, 0.0)
        w = dv * (r_ref[0:1, :] + r_ref[1:2, :]) + dv * dv
        if masked:
            glob = lax.broadcasted_iota(jnp.int32, (1, RL), 1) + (H + i * RL)
            w = jnp.where(glob < n_nodes, w, 0.0)
        return jnp.sum(h1t * w, axis=1, keepdims=True)

    acc_ref[...] += (half(a0_ref, dinv0_ref, x0_ref, r_0ref, False)
                     + half(a1_ref, dinv1_ref, x1_ref, r_1ref, True))

    @pl.when(i == n_steps - 1)
    def _head():
        nf = jnp.float32(n_nodes)
        s_t = acc_ref[...]
        e_t = lax.dot_general(w2_ref[...], s_t, (((0,), (0,)), ((), ())),
                              preferred_element_type=jnp.float32) + nf * b2_ref[...]
        out_ref[...] = lax.dot_general(wl_ref[...], e_t, (((0,), (0,)), ((), ())),
                                       preferred_element_type=jnp.float32
                                       ) + nf * bl_ref[...]


def kernel(x, edge_index, edge_attr, emb, W1, b1, W2, b2, Wl, bl):
    n = x.shape[0]
    e = edge_index.shape[1]
    Np = _pad_up(n, 2048)
    Ep = _pad_up(e, 32 * CH)
    H = Np // 2
    A_SIZE = H * NCLS

    src = edge_index[0].astype(jnp.int32)
    dst = edge_index[1].astype(jnp.int32)
    pad_idx = jnp.full((Ep - e,), Np - 1, jnp.int32)
    src_p = jnp.concatenate([src, pad_idx]).reshape(Ep // LANES, LANES)
    dst_p = jnp.concatenate([dst, pad_idx]).reshape(Ep // LANES, LANES)
    xcls = jnp.concatenate(
        [x[:, 0].astype(jnp.int32), jnp.zeros((Np - n,), jnp.int32)])

    zeros_big = jnp.zeros(((A_SIZE + 2048) // 16,), jnp.float32)
    ones_chunk = jnp.ones((KR, LANES), jnp.float32)

    deg2 = _make_deg_kernel(Ep, Np)(dst_p, zeros_big, ones_chunk)

    dinv_2d, q_2d = pl.pallas_call(
        _rsqrt_body,
        out_shape=(jax.ShapeDtypeStruct((Np // 128, 128), jnp.float32),
                   jax.ShapeDtypeStruct((Np // 128, 128), jnp.float32)),
    )(deg2.reshape(2, Np // 128, 128), xcls.reshape(Np // 128, 128))
    dinv = dinv_2d.reshape(Np)
    q_tab = q_2d.reshape(Np)

    a_halves, r2 = _make_main_kernel(Ep, Np)(src_p, dst_p, q_tab, zeros_big)
    a0 = a_halves[0].reshape(NCLS, H)
    a1 = a_halves[1].reshape(NCLS, H)

    n_steps = 8
    RL = H // n_steps
    dinv_row = dinv.reshape(1, Np)
    x_row = xcls.reshape(1, Np)
    lo = lambda i: (0, i)
    hi = lambda i: (0, n_steps + i)
    head = pl.pallas_call(
        functools.partial(_dense_body, n, n_steps, RL, H),
        grid=(n_steps,),
        in_specs=[
            pl.BlockSpec((NCLS, RL), lo),
            pl.BlockSpec((NCLS, RL), lambda i: (0, i)),
            pl.BlockSpec((1, RL), lo),
            pl.BlockSpec((1, RL), hi),
            pl.BlockSpec((1, RL), lo),
            pl.BlockSpec((1, RL), hi),
            pl.BlockSpec((2, RL), lo),
            pl.BlockSpec((2, RL), hi),
            pl.BlockSpec((NCLS, 16), lambda i: (0, 0)),
            pl.BlockSpec((16, 16), lambda i: (0, 0)),
            pl.BlockSpec((16, 1), lambda i: (0, 0)),
            pl.BlockSpec((16, 16), lambda i: (0, 0)),
            pl.BlockSpec((16, 1), lambda i: (0, 0)),
            pl.BlockSpec((16, 1), lambda i: (0, 0)),
            pl.BlockSpec((1, 1), lambda i: (0, 0)),
        ],
        out_specs=pl.BlockSpec((1, 1), lambda i: (0, 0)),
        out_shape=jax.ShapeDtypeStruct((1, 1), jnp.float32),
        scratch_shapes=[pltpu.VMEM((16, 1), jnp.float32)],
    )(a0, a1, dinv_row, dinv_row, x_row, x_row, r2, r2,
      emb, W1, b1.reshape(16, 1), W2, b2.reshape(16, 1), Wl,
      bl.reshape(1, 1))

    return head.reshape(1)
